# baseline (speedup 1.0000x reference)
import functools

import jax
import jax.numpy as jnp
from jax import lax
from jax.experimental import pallas as pl
from jax.experimental.pallas import tpu as pltpu
from jax.experimental.pallas import tpu_sc as plsc

N = 10000
E = 320000
D = 128
H = 16
C = 2

NC = 2
NS = 16
NW = NC * NS
CH = 128
NCH = 80
NBUF = 8
NB = NCH // NBUF
EPT = NCH * CH
TOT = NW * EPT
NP = 10112
RPT = NP // NS

_mesh = plsc.VectorSubcoreMesh(core_axis_name="c", subcore_axis_name="s")
_sc_params = pltpu.CompilerParams(use_tc_tiling_on_sc=False)


@functools.partial(
    pl.kernel,
    out_type=jax.ShapeDtypeStruct((NC, NP), jnp.float32),
    mesh=_mesh,
    compiler_params=_sc_params,
    scratch_types=[
        pltpu.VMEM((NCH, CH), jnp.int32),
        pltpu.VMEM((CH,), jnp.float32),
        pltpu.VMEM((RPT,), jnp.float32),
        pltpu.VMEM_SHARED((NP,), jnp.float32),
    ],
)
def _deg_sc(dst_hbm, zeros1_hbm, out_hbm,
            dst_v, ones_v, deg_v, acc_sh):
    c = lax.axis_index("c")
    s = lax.axis_index("s")
    wid = s * NC + c
    r0 = s * RPT
    pltpu.sync_copy(zeros1_hbm.at[pl.ds(r0, RPT)], deg_v)
    pltpu.sync_copy(deg_v, acc_sh.at[pl.ds(r0, RPT)])
    pltpu.sync_copy(dst_hbm.at[wid], dst_v)
    for i in range(CH // 16):
        ones_v[pl.ds(16 * i, 16)] = jnp.ones((16,), jnp.float32)
    plsc.subcore_barrier()

    def body(j, carry):
        pltpu.sync_copy(ones_v, acc_sh.at[dst_v.at[j]], add=True)
        return carry

    lax.fori_loop(0, NCH, body, 0)
    plsc.subcore_barrier()

    pltpu.sync_copy(acc_sh.at[pl.ds(r0, RPT)], deg_v)
    pltpu.sync_copy(deg_v, out_hbm.at[c].at[pl.ds(r0, RPT)])


@functools.partial(
    pl.kernel,
    out_type=jax.ShapeDtypeStruct((NC, NP, H), jnp.float32),
    mesh=_mesh,
    compiler_params=_sc_params,
    scratch_types=[
        pltpu.VMEM((NCH, CH), jnp.int32),
        pltpu.VMEM((NCH, CH), jnp.int32),
        pltpu.VMEM((NBUF, CH, H), jnp.float32),
        pltpu.VMEM((RPT, H), jnp.float32),
        pltpu.VMEM_SHARED((NP, H), jnp.float32),
        pltpu.SemaphoreType.DMA((NBUF,)),
        pltpu.SemaphoreType.DMA((NBUF,)),
    ],
)
def _agg_sc(g_hbm, src_hbm, dst_hbm, zeros2_hbm, out_hbm,
            src_v, dst_v, rows_v, stage_v, acc_sh, gsem, ssem):
    c = lax.axis_index("c")
    s = lax.axis_index("s")
    wid = s * NC + c
    r0 = s * RPT
    pltpu.sync_copy(zeros2_hbm.at[pl.ds(r0, RPT)], stage_v)
    pltpu.sync_copy(stage_v, acc_sh.at[pl.ds(r0, RPT)])
    pltpu.sync_copy(src_hbm.at[wid], src_v)
    pltpu.sync_copy(dst_hbm.at[wid], dst_v)
    for b in range(NBUF):
        pltpu.async_copy(g_hbm.at[src_v.at[b]], rows_v.at[b], gsem.at[b])
    plsc.subcore_barrier()

    def block(k, carry):
        scats = []
        for b in range(NBUF):
            j = k * NBUF + b
            pltpu.make_async_copy(
                g_hbm.at[src_v.at[j]], rows_v.at[b], gsem.at[b]).wait()
            scats.append(pltpu.async_copy(
                rows_v.at[b], acc_sh.at[dst_v.at[j]], ssem.at[b], add=True))
        @pl.when(k + 1 < NB)
        def _issue_next():
            for b in range(NBUF):
                scats[b].wait()
                jn = (k + 1) * NBUF + b
                pltpu.async_copy(g_hbm.at[src_v.at[jn]], rows_v.at[b],
                                 gsem.at[b])

        @pl.when(k + 1 >= NB)
        def _drain_last():
            for b in range(NBUF):
                scats[b].wait()
        return carry

    lax.fori_loop(0, NB, block, 0)
    plsc.subcore_barrier()
    pltpu.sync_copy(acc_sh.at[pl.ds(r0, RPT)], stage_v)
    pltpu.sync_copy(stage_v, out_hbm.at[c].at[pl.ds(r0, RPT)])


def _tc1_body(x_ref, w1_ref, degw_ref, g1_ref, dinv_ref):
    deg = degw_ref[0] + degw_ref[1]
    dinv = lax.rsqrt(deg[:N] + 1.0)
    h = jnp.dot(x_ref[...], w1_ref[...],
                preferred_element_type=jnp.float32)
    g1_ref[...] = h * dinv
    dinv_ref[...] = dinv


def _tc2_body(s1_ref, g1_ref, dinv_ref, b1_ref, g2_ref):
    dinv = dinv_ref[...]
    z = (s1_ref[0, :N] + s1_ref[1, :N] + g1_ref[...]) * dinv + b1_ref[...]
    g2_ref[...] = jnp.maximum(z, 0.0) * dinv


def _tc3_body(s2_ref, g2_ref, dinv_ref, w2_ref, b2_ref, out_ref):
    u = (s2_ref[0, :N] + s2_ref[1, :N] + g2_ref[...]) * dinv_ref[...]
    y = jnp.dot(u, w2_ref[...],
                preferred_element_type=jnp.float32) + b2_ref[...]
    m = jnp.max(y, axis=1, keepdims=True)
    lse = m + jnp.log(jnp.sum(jnp.exp(y - m), axis=1, keepdims=True))
    out_ref[...] = y - lse


_tc1 = pl.pallas_call(
    _tc1_body,
    out_shape=(jax.ShapeDtypeStruct((N, H), jnp.float32),
               jax.ShapeDtypeStruct((N, 1), jnp.float32)),
)
_tc2 = pl.pallas_call(
    _tc2_body,
    out_shape=jax.ShapeDtypeStruct((N, H), jnp.float32),
)
_tc3 = pl.pallas_call(
    _tc3_body,
    out_shape=jax.ShapeDtypeStruct((N, C), jnp.float32),
)


def kernel(x, edge_index, W1, b1, W2, b2):
    src = edge_index[0]
    dst = edge_index[1]
    padn = TOT - E
    srcp = jnp.concatenate(
        [src, jnp.zeros((padn,), src.dtype)]).reshape(NW, NCH, CH)
    dstp = jnp.concatenate(
        [dst, jnp.full((padn,), N, dst.dtype)]).reshape(NW, NCH, CH)
    zeros1 = jnp.zeros((NP,), jnp.float32)
    zeros2 = jnp.zeros((NP, H), jnp.float32)

    degw = _deg_sc(dstp, zeros1).reshape(NC, NP, 1)
    g1, dinv = _tc1(x, W1, degw)
    s1 = _agg_sc(g1, srcp, dstp, zeros2)
    g2 = _tc2(s1, g1, dinv, b1.reshape(1, H))
    s2 = _agg_sc(g2, srcp, dstp, zeros2)
    out = _tc3(s2, g2, dinv, W2, b2.reshape(1, C))
    return out


# speedup vs baseline: 1.0176x; 1.0176x over previous
import functools

import jax
import jax.numpy as jnp
from jax import lax
from jax.experimental import pallas as pl
from jax.experimental.pallas import tpu as pltpu
from jax.experimental.pallas import tpu_sc as plsc

N = 10000
E = 320000
D = 128
H = 16
C = 2

NC = 2
NS = 16
NW = NC * NS
CH = 128
NCH = 80
NBUF = 8
NB = NCH // NBUF
EPT = NCH * CH
TOT = NW * EPT
NP = 10112
RPT = NP // NS

_mesh = plsc.VectorSubcoreMesh(core_axis_name="c", subcore_axis_name="s")
_sc_params = pltpu.CompilerParams(use_tc_tiling_on_sc=False)


@functools.partial(
    pl.kernel,
    out_type=jax.ShapeDtypeStruct((NC, NP), jnp.float32),
    mesh=_mesh,
    compiler_params=_sc_params,
    scratch_types=[
        pltpu.VMEM((NCH, CH), jnp.int32),
        pltpu.VMEM((CH,), jnp.float32),
        pltpu.VMEM((RPT,), jnp.float32),
        pltpu.VMEM_SHARED((NP,), jnp.float32),
    ],
)
def _deg_sc(dst_hbm, zeros1_hbm, out_hbm,
            dst_v, ones_v, deg_v, acc_sh):
    c = lax.axis_index("c")
    s = lax.axis_index("s")
    wid = s * NC + c
    r0 = s * RPT
    pltpu.sync_copy(zeros1_hbm.at[pl.ds(r0, RPT)], deg_v)
    pltpu.sync_copy(deg_v, acc_sh.at[pl.ds(r0, RPT)])
    pltpu.sync_copy(dst_hbm.at[wid], dst_v)
    for i in range(CH // 16):
        ones_v[pl.ds(16 * i, 16)] = jnp.ones((16,), jnp.float32)
    plsc.subcore_barrier()

    def body(j, carry):
        pltpu.sync_copy(ones_v, acc_sh.at[dst_v.at[j]], add=True)
        return carry

    lax.fori_loop(0, NCH, body, 0)
    plsc.subcore_barrier()

    pltpu.sync_copy(acc_sh.at[pl.ds(r0, RPT)], deg_v)
    pltpu.sync_copy(deg_v, out_hbm.at[c].at[pl.ds(r0, RPT)])


def _make_agg(W):
    @functools.partial(
        pl.kernel,
        out_type=jax.ShapeDtypeStruct((NC, NP, W), jnp.float32),
        mesh=_mesh,
        compiler_params=_sc_params,
        scratch_types=[
            pltpu.VMEM((NCH, CH), jnp.int32),
            pltpu.VMEM((NCH, CH), jnp.int32),
            pltpu.VMEM((NBUF, CH, W), jnp.float32),
            pltpu.VMEM((RPT, W), jnp.float32),
            pltpu.VMEM_SHARED((NP, W), jnp.float32),
            pltpu.SemaphoreType.DMA((NBUF,)),
            pltpu.SemaphoreType.DMA((NBUF,)),
        ],
    )
    def agg(g_hbm, src_hbm, dst_hbm, zeros_hbm, out_hbm,
            src_v, dst_v, rows_v, stage_v, acc_sh, gsem, ssem):
        c = lax.axis_index("c")
        s = lax.axis_index("s")
        wid = s * NC + c
        r0 = s * RPT
        pltpu.sync_copy(zeros_hbm.at[pl.ds(r0, RPT)], stage_v)
        pltpu.sync_copy(stage_v, acc_sh.at[pl.ds(r0, RPT)])
        pltpu.sync_copy(src_hbm.at[wid], src_v)
        pltpu.sync_copy(dst_hbm.at[wid], dst_v)
        for b in range(NBUF):
            pltpu.async_copy(g_hbm.at[src_v.at[b]], rows_v.at[b], gsem.at[b])
        plsc.subcore_barrier()

        def block(k, carry):
            scats = []
            for b in range(NBUF):
                j = k * NBUF + b
                pltpu.make_async_copy(
                    g_hbm.at[src_v.at[j]], rows_v.at[b], gsem.at[b]).wait()
                scats.append(pltpu.async_copy(
                    rows_v.at[b], acc_sh.at[dst_v.at[j]], ssem.at[b],
                    add=True))

            @pl.when(k + 1 < NB)
            def _issue_next():
                for b in range(NBUF):
                    scats[b].wait()
                    jn = (k + 1) * NBUF + b
                    pltpu.async_copy(g_hbm.at[src_v.at[jn]], rows_v.at[b],
                                     gsem.at[b])

            @pl.when(k + 1 >= NB)
            def _drain_last():
                for b in range(NBUF):
                    scats[b].wait()
            return carry

        lax.fori_loop(0, NB, block, 0)
        plsc.subcore_barrier()
        pltpu.sync_copy(acc_sh.at[pl.ds(r0, RPT)], stage_v)
        pltpu.sync_copy(stage_v, out_hbm.at[c].at[pl.ds(r0, RPT)])

    return agg


_agg16 = _make_agg(H)
_agg2 = _make_agg(C)


def _tc1_body(x_ref, w1_ref, degw_ref, g1_ref, dinv_ref):
    deg = degw_ref[0] + degw_ref[1]
    dinv = lax.rsqrt(deg[:N] + 1.0)
    h = jnp.dot(x_ref[...], w1_ref[...],
                preferred_element_type=jnp.float32)
    g1_ref[...] = h * dinv
    dinv_ref[...] = dinv


def _tc2_body(s1_ref, g1_ref, dinv_ref, b1_ref, w2_ref, y_ref):
    dinv = dinv_ref[...]
    z = (s1_ref[0, :N] + s1_ref[1, :N] + g1_ref[...]) * dinv + b1_ref[...]
    y_ref[...] = jnp.dot(jnp.maximum(z, 0.0) * dinv, w2_ref[...],
                         preferred_element_type=jnp.float32)


def _tc3_body(s2_ref, y_ref, dinv_ref, b2_ref, out_ref):
    y = (s2_ref[0, :N] + s2_ref[1, :N] + y_ref[...]) * dinv_ref[...] \
        + b2_ref[...]
    m = jnp.max(y, axis=1, keepdims=True)
    lse = m + jnp.log(jnp.sum(jnp.exp(y - m), axis=1, keepdims=True))
    out_ref[...] = y - lse


_tc1 = pl.pallas_call(
    _tc1_body,
    out_shape=(jax.ShapeDtypeStruct((N, H), jnp.float32),
               jax.ShapeDtypeStruct((N, 1), jnp.float32)),
)
_tc2 = pl.pallas_call(
    _tc2_body,
    out_shape=jax.ShapeDtypeStruct((N, C), jnp.float32),
)
_tc3 = pl.pallas_call(
    _tc3_body,
    out_shape=jax.ShapeDtypeStruct((N, C), jnp.float32),
)


def kernel(x, edge_index, W1, b1, W2, b2):
    src = edge_index[0]
    dst = edge_index[1]
    padn = TOT - E
    srcp = jnp.concatenate(
        [src, jnp.zeros((padn,), src.dtype)]).reshape(NW, NCH, CH)
    dstp = jnp.concatenate(
        [dst, jnp.full((padn,), N, dst.dtype)]).reshape(NW, NCH, CH)
    zeros1 = jnp.zeros((NP,), jnp.float32)
    zeros2 = jnp.zeros((NP, H), jnp.float32)
    zerosc = jnp.zeros((NP, C), jnp.float32)

    degw = _deg_sc(dstp, zeros1).reshape(NC, NP, 1)
    g1, dinv = _tc1(x, W1, degw)
    s1 = _agg16(g1, srcp, dstp, zeros2)
    y = _tc2(s1, g1, dinv, b1.reshape(1, H), W2)
    s2 = _agg2(y, srcp, dstp, zerosc)
    out = _tc3(s2, y, dinv, b2.reshape(1, C))
    return out
